# baseline (speedup 1.0000x reference)
import functools

import jax
import jax.numpy as jnp
from jax.experimental import pallas as pl
from jax.experimental.pallas import tpu as pltpu

_ROWS = 256


def _trim_body(x_ref, v_ref, mask_ref, xo_ref, vo_ref, mo_ref, maxlen_ref):
    i = pl.program_id(0)
    L = x_ref.shape[-1]

    @pl.when(i == 0)
    def _prologue():
        m = mask_ref[...]
        maxlen = jnp.maximum(jnp.max(jnp.sum(m, axis=-1)), 1)
        maxlen_ref[0] = maxlen
        keep_row = jax.lax.broadcasted_iota(jnp.int32, (1, L), 1) < maxlen
        mo_ref[...] = jnp.where(keep_row, m, 0)
        vo_ref[...] = jnp.where(keep_row, v_ref[...], 0.0)

    maxlen = maxlen_ref[0]
    keep = jax.lax.broadcasted_iota(jnp.int32, x_ref.shape, 1) < maxlen
    xo_ref[...] = jnp.where(keep, x_ref[...], 0.0)


@functools.partial(jax.jit, static_argnames=())
def kernel(x, v, mask):
    B, C, L = x.shape
    Cv = v.shape[1]
    x2 = x.reshape(B * C, L)
    v2 = v.reshape(B * Cv, L)
    m2 = mask.reshape(B, L)
    n_blocks = (B * C) // _ROWS

    x_out2, v_out2, m_out2 = pl.pallas_call(
        _trim_body,
        grid=(n_blocks,),
        in_specs=[
            pl.BlockSpec((_ROWS, L), lambda i: (i, 0)),
            pl.BlockSpec((B * Cv, L), lambda i: (0, 0)),
            pl.BlockSpec((B, L), lambda i: (0, 0)),
        ],
        out_specs=[
            pl.BlockSpec((_ROWS, L), lambda i: (i, 0)),
            pl.BlockSpec((B * Cv, L), lambda i: (0, 0)),
            pl.BlockSpec((B, L), lambda i: (0, 0)),
        ],
        out_shape=[
            jax.ShapeDtypeStruct((B * C, L), x.dtype),
            jax.ShapeDtypeStruct((B * Cv, L), v.dtype),
            jax.ShapeDtypeStruct((B, L), jnp.int32),
        ],
        scratch_shapes=[pltpu.SMEM((1,), jnp.int32)],
    )(x2, v2, m2)

    return (
        x_out2.reshape(B, C, L),
        v_out2.reshape(B, Cv, L),
        m_out2.reshape(B, 1, L).astype(bool),
    )
